# baseline (device time: 12410 ns/iter reference)
import jax
import jax.numpy as jnp
from jax import lax
from jax.experimental import pallas as pl
from jax.experimental.pallas import tpu as pltpu

N_DEV = 4


def kernel(x, dy, gamma):
    m, d = x.shape

    def body(x_ref, dy_ref, gamma_ref, out_ref, comm_ref, send_sems, recv_sems):
        my_pos = lax.axis_index("i")
        left = (my_pos - 1) % N_DEV
        right = (my_pos + 1) % N_DEV

        barrier_sem = pltpu.get_barrier_semaphore()
        for nbr in (left, right):
            pl.semaphore_signal(
                barrier_sem, inc=1,
                device_id=(nbr,), device_id_type=pl.DeviceIdType.MESH,
            )
        pl.semaphore_wait(barrier_sem, 2)

        xv = x_ref[:, :]
        dyv = dy_ref[:, :]
        mu = jnp.mean(xv, axis=1, keepdims=True)
        xc = xv - mu
        var = jnp.mean(xc * xc, axis=1, keepdims=True)
        rstd = lax.rsqrt(var + 1e-5)
        pdgamma = jnp.sum(dyv * (xc * rstd), axis=0, keepdims=True)
        pdbeta = jnp.sum(dyv, axis=0, keepdims=True)

        comm_ref[0, 0:1, :] = pdgamma
        comm_ref[0, 1:2, :] = pdbeta
        out_ref[0:1, :] = pdgamma
        out_ref[1:2, :] = pdbeta

        for h in range(N_DEV - 1):
            rdma = pltpu.make_async_remote_copy(
                src_ref=comm_ref.at[h],
                dst_ref=comm_ref.at[h + 1],
                send_sem=send_sems.at[h],
                recv_sem=recv_sems.at[h],
                device_id=(right,),
                device_id_type=pl.DeviceIdType.MESH,
            )
            rdma.start()
            rdma.wait()
            out_ref[:, :] = out_ref[:, :] + comm_ref[h + 1, :, :]

    return pl.pallas_call(
        body,
        out_shape=jax.ShapeDtypeStruct((2, d), jnp.float32),
        in_specs=[
            pl.BlockSpec(memory_space=pltpu.VMEM),
            pl.BlockSpec(memory_space=pltpu.VMEM),
            pl.BlockSpec(memory_space=pltpu.VMEM),
        ],
        out_specs=pl.BlockSpec(memory_space=pltpu.VMEM),
        scratch_shapes=[
            pltpu.VMEM((N_DEV, 2, d), jnp.float32),
            pltpu.SemaphoreType.DMA((N_DEV - 1,)),
            pltpu.SemaphoreType.DMA((N_DEV - 1,)),
        ],
        compiler_params=pltpu.CompilerParams(collective_id=0),
    )(x, dy, gamma)


# device time: 8798 ns/iter; 1.4105x vs baseline; 1.4105x over previous
import jax
import jax.numpy as jnp
from jax import lax
from jax.experimental import pallas as pl
from jax.experimental.pallas import tpu as pltpu

N_DEV = 4


def kernel(x, dy, gamma):
    m, d = x.shape

    def body(x_ref, dy_ref, gamma_ref, out_ref, comm_ref, send_sems, recv_sems):
        my_pos = lax.axis_index("i")

        barrier_sem = pltpu.get_barrier_semaphore()
        for k in range(1, N_DEV):
            pl.semaphore_signal(
                barrier_sem, inc=1,
                device_id=((my_pos + k) % N_DEV,),
                device_id_type=pl.DeviceIdType.MESH,
            )

        xv = x_ref[:, :]
        dyv = dy_ref[:, :]
        mu = jnp.mean(xv, axis=1, keepdims=True)
        xc = xv - mu
        var = jnp.mean(xc * xc, axis=1, keepdims=True)
        rstd = lax.rsqrt(var + 1e-5)
        pdgamma = jnp.sum(dyv * (xc * rstd), axis=0, keepdims=True)
        pdbeta = jnp.sum(dyv, axis=0, keepdims=True)
        local = jnp.concatenate([pdgamma, pdbeta], axis=0)
        send_ref = comm_ref.at[N_DEV - 1]
        send_ref[:, :] = local

        pl.semaphore_wait(barrier_sem, N_DEV - 1)

        rdmas = []
        for k in range(1, N_DEV):
            rdma = pltpu.make_async_remote_copy(
                src_ref=send_ref,
                dst_ref=comm_ref.at[N_DEV - 1 - k],
                send_sem=send_sems.at[k - 1],
                recv_sem=recv_sems.at[N_DEV - 1 - k],
                device_id=((my_pos + k) % N_DEV,),
                device_id_type=pl.DeviceIdType.MESH,
            )
            rdma.start()
            rdmas.append(rdma)

        for r in range(N_DEV - 1):
            recv = pltpu.make_async_remote_copy(
                src_ref=send_ref,
                dst_ref=comm_ref.at[r],
                send_sem=send_sems.at[0],
                recv_sem=recv_sems.at[r],
                device_id=(my_pos,),
                device_id_type=pl.DeviceIdType.MESH,
            )
            recv.wait_recv()
        out_ref[:, :] = (
            local + comm_ref[0, :, :] + comm_ref[1, :, :] + comm_ref[2, :, :]
        )

        for rdma in rdmas:
            rdma.wait_send()

    return pl.pallas_call(
        body,
        out_shape=jax.ShapeDtypeStruct((2, d), jnp.float32),
        in_specs=[
            pl.BlockSpec(memory_space=pltpu.VMEM),
            pl.BlockSpec(memory_space=pltpu.VMEM),
            pl.BlockSpec(memory_space=pltpu.VMEM),
        ],
        out_specs=pl.BlockSpec(memory_space=pltpu.VMEM),
        scratch_shapes=[
            pltpu.VMEM((N_DEV, 2, d), jnp.float32),
            pltpu.SemaphoreType.DMA((N_DEV - 1,)),
            pltpu.SemaphoreType.DMA((N_DEV - 1,)),
        ],
        compiler_params=pltpu.CompilerParams(collective_id=0),
    )(x, dy, gamma)
